# baseline (speedup 1.0000x reference)
import jax
import jax.numpy as jnp
from jax.experimental import pallas as pl
from jax.experimental.pallas import tpu as pltpu

_K = 10
_CW = 500
_TPC = 4
_FILL = float(jnp.finfo(jnp.float32).min)
_NEG = float("-inf")
_IDBITS = 17
_BIGI = 1 << 30


def _topk_ids_naive(x, giota, n):
    work = x
    ids = []
    for _ in range(_K):
        mx = jnp.max(work, axis=(1, 2), keepdims=True)
        am = jnp.min(jnp.where(work == mx, giota, n), axis=(1, 2), keepdims=True)
        ids.append(am[:, 0])
        work = jnp.where(giota == am, _NEG, work)
    return jnp.concatenate(ids, axis=1)


def _topk_ids_hier(x, giota, n):
    work = x
    cvals = []
    cids = []
    for _ in range(_TPC):
        mx = jnp.max(work, axis=2, keepdims=True)
        am = jnp.min(jnp.where(work == mx, giota, n), axis=2, keepdims=True)
        cvals.append(mx)
        cids.append(am)
        work = jnp.where(giota == am, _NEG, work)
    cv = jnp.concatenate(cvals, axis=2)
    ci = jnp.concatenate(cids, axis=2)
    kidx = jax.lax.broadcasted_iota(jnp.int32, cv.shape, 2)
    ids = []
    unsafe = jnp.zeros((cv.shape[0], 1, 1), dtype=jnp.bool_)
    for _ in range(_K):
        mx = jnp.max(cv, axis=(1, 2), keepdims=True)
        am = jnp.min(jnp.where(cv == mx, ci, n), axis=(1, 2), keepdims=True)
        ids.append(am[:, 0])
        hit = jnp.logical_and(cv == mx, ci == am)
        unsafe = jnp.logical_or(
            unsafe,
            jnp.any(jnp.logical_and(hit, kidx == _TPC - 1), axis=(1, 2), keepdims=True),
        )
        cv = jnp.where(hit, _NEG, cv)
    return jnp.concatenate(ids, axis=1), unsafe


def _rec_kernel(w_ref, r_ref, out_ref):
    RB, C, W = out_ref.shape
    n = C * W
    x0 = r_ref[0]
    x1 = r_ref[1]
    giota = (
        jax.lax.broadcasted_iota(jnp.int32, (RB, C, W), 1) * W
        + jax.lax.broadcasted_iota(jnp.int32, (RB, C, W), 2)
    )

    h0, u0 = _topk_ids_hier(x0, giota, n)
    h1, u1 = _topk_ids_hier(x1, giota, n)
    unsafe = jnp.any(jnp.logical_or(u0, u1))

    ids0, ids1 = jax.lax.cond(
        unsafe,
        lambda: (_topk_ids_naive(x0, giota, n), _topk_ids_naive(x1, giota, n)),
        lambda: (h0, h1),
    )

    eq = ids0[:, :, None] == ids1[:, None, :]
    dup0 = jnp.sum(eq.astype(jnp.int32), axis=2)
    dup1 = jnp.sum(eq.astype(jnp.int32), axis=1)
    cand_ids = jnp.concatenate([ids0, ids1], axis=1)
    cnt = jnp.concatenate([1 + dup0, 1 + dup1], axis=1)
    valid = jnp.concatenate([jnp.ones_like(dup1), 1 - dup1], axis=1)
    key = jnp.where(valid > 0, (cnt << _IDBITS) - cand_ids, -_BIGI)
    jidx = jax.lax.broadcasted_iota(jnp.int32, key.shape, 1)

    w0 = w_ref[0, 0]
    w1 = w_ref[0, 1]
    xc = w0 * x0 + w1 * x1

    selmask = jnp.zeros((RB, C, W), dtype=jnp.bool_)
    for _ in range(_K):
        mk = jnp.max(key, axis=1, keepdims=True)
        amj = jnp.min(jnp.where(key == mk, jidx, 2 * _K), axis=1, keepdims=True)
        sel = jnp.sum(jnp.where(jidx == amj, cand_ids, 0), axis=1, keepdims=True)
        selmask = jnp.logical_or(selmask, giota == sel[:, :, None])
        key = jnp.where(jidx == amj, -_BIGI, key)

    out_ref[...] = jnp.where(selmask, xc, _FILL)


def kernel(ratings, w):
    M, B, N = ratings.shape
    RB = 8
    C = N // _CW
    r4 = ratings.reshape(M, B, C, _CW)
    w2 = w.reshape(1, M).astype(jnp.float32)
    out = pl.pallas_call(
        _rec_kernel,
        grid=(B // RB,),
        in_specs=[
            pl.BlockSpec(memory_space=pltpu.SMEM),
            pl.BlockSpec((M, RB, C, _CW), lambda i: (0, i, 0, 0)),
        ],
        out_specs=pl.BlockSpec((RB, C, _CW), lambda i: (i, 0, 0)),
        out_shape=jax.ShapeDtypeStruct((B, C, _CW), jnp.float32),
    )(w2, r4)
    return out.reshape(B, N)


# speedup vs baseline: 1.7388x; 1.7388x over previous
import jax
import jax.numpy as jnp
from jax.experimental import pallas as pl
from jax.experimental.pallas import tpu as pltpu

_K = 10
_CW = 125
_TPC = 4
_FILL = float(jnp.finfo(jnp.float32).min)
_NEG = float("-inf")
_IDBITS = 17
_BIGI = 1 << 30


def _topk_ids_naive(x, giota, n):
    work = x
    ids = []
    for _ in range(_K):
        mx = jnp.max(work, axis=(1, 2), keepdims=True)
        am = jnp.min(jnp.where(work == mx, giota, n), axis=(1, 2), keepdims=True)
        ids.append(am[:, 0])
        work = jnp.where(giota == am, _NEG, work)
    return jnp.concatenate(ids, axis=1)


def _topk_ids_hier(x, giota, n):
    work = x
    cvals = []
    cids = []
    for _ in range(_TPC):
        mx = jnp.max(work, axis=1, keepdims=True)
        am = jnp.min(jnp.where(work == mx, giota, n), axis=1, keepdims=True)
        cvals.append(mx)
        cids.append(am)
        work = jnp.where(giota == am, _NEG, work)
    cv = jnp.concatenate(cvals, axis=1)
    ci = jnp.concatenate(cids, axis=1)
    kidx = jax.lax.broadcasted_iota(jnp.int32, cv.shape, 1)
    ids = []
    unsafe = jnp.zeros((cv.shape[0], 1, 1), dtype=jnp.bool_)
    for _ in range(_K):
        mx = jnp.max(cv, axis=(1, 2), keepdims=True)
        am = jnp.min(jnp.where(cv == mx, ci, n), axis=(1, 2), keepdims=True)
        ids.append(am[:, 0])
        hit = jnp.logical_and(cv == mx, ci == am)
        unsafe = jnp.logical_or(
            unsafe,
            jnp.any(jnp.logical_and(hit, kidx == _TPC - 1), axis=(1, 2), keepdims=True),
        )
        cv = jnp.where(hit, _NEG, cv)
    return jnp.concatenate(ids, axis=1), unsafe


def _rec_kernel(w_ref, r_ref, out_ref):
    RB, C, W = out_ref.shape
    n = C * W
    x0 = r_ref[0]
    x1 = r_ref[1]
    giota = (
        jax.lax.broadcasted_iota(jnp.int32, (RB, C, W), 1) * W
        + jax.lax.broadcasted_iota(jnp.int32, (RB, C, W), 2)
    )

    h0, u0 = _topk_ids_hier(x0, giota, n)
    h1, u1 = _topk_ids_hier(x1, giota, n)
    unsafe = jnp.any(jnp.logical_or(u0, u1))

    ids0, ids1 = jax.lax.cond(
        unsafe,
        lambda: (_topk_ids_naive(x0, giota, n), _topk_ids_naive(x1, giota, n)),
        lambda: (h0, h1),
    )

    eq = ids0[:, :, None] == ids1[:, None, :]
    dup0 = jnp.sum(eq.astype(jnp.int32), axis=2)
    dup1 = jnp.sum(eq.astype(jnp.int32), axis=1)
    cand_ids = jnp.concatenate([ids0, ids1], axis=1)
    cnt = jnp.concatenate([1 + dup0, 1 + dup1], axis=1)
    valid = jnp.concatenate([jnp.ones_like(dup1), 1 - dup1], axis=1)
    key = jnp.where(valid > 0, (cnt << _IDBITS) - cand_ids, -_BIGI)
    jidx = jax.lax.broadcasted_iota(jnp.int32, key.shape, 1)

    w0 = w_ref[0, 0]
    w1 = w_ref[0, 1]
    xc = w0 * x0 + w1 * x1

    selmask = jnp.zeros((RB, C, W), dtype=jnp.bool_)
    for _ in range(_K):
        mk = jnp.max(key, axis=1, keepdims=True)
        amj = jnp.min(jnp.where(key == mk, jidx, 2 * _K), axis=1, keepdims=True)
        sel = jnp.sum(jnp.where(jidx == amj, cand_ids, 0), axis=1, keepdims=True)
        selmask = jnp.logical_or(selmask, giota == sel[:, :, None])
        key = jnp.where(jidx == amj, -_BIGI, key)

    out_ref[...] = jnp.where(selmask, xc, _FILL)


def kernel(ratings, w):
    M, B, N = ratings.shape
    RB = 8
    C = N // _CW
    r4 = ratings.reshape(M, B, C, _CW)
    w2 = w.reshape(1, M).astype(jnp.float32)
    out = pl.pallas_call(
        _rec_kernel,
        grid=(B // RB,),
        in_specs=[
            pl.BlockSpec(memory_space=pltpu.SMEM),
            pl.BlockSpec((M, RB, C, _CW), lambda i: (0, i, 0, 0)),
        ],
        out_specs=pl.BlockSpec((RB, C, _CW), lambda i: (i, 0, 0)),
        out_shape=jax.ShapeDtypeStruct((B, C, _CW), jnp.float32),
    )(w2, r4)
    return out.reshape(B, N)


# speedup vs baseline: 1.8674x; 1.0739x over previous
import jax
import jax.numpy as jnp
from jax.experimental import pallas as pl
from jax.experimental.pallas import tpu as pltpu

_K = 10
_L = 125
_D = 3
_FILL = float(jnp.finfo(jnp.float32).min)
_NEG = float("-inf")
_IDBITS = 17
_BIGI = 1 << 30


def _topk_ids_naive(x, giota, n):
    work = x
    ids = []
    for _ in range(_K):
        mx = jnp.max(work, axis=(1, 2), keepdims=True)
        am = jnp.min(jnp.where(work == mx, giota, n), axis=(1, 2), keepdims=True)
        ids.append(am[:, 0])
        work = jnp.where(giota == am, _NEG, work)
    return jnp.concatenate(ids, axis=1)


def _online_top3(r_ref, m, RB):
    S, L = r_ref.shape[2], r_ref.shape[3]
    G = S // 8
    base = (
        jax.lax.broadcasted_iota(jnp.int32, (RB, 8, L), 1) * L
        + jax.lax.broadcasted_iota(jnp.int32, (RB, 8, L), 2)
    )
    neg = jnp.full((RB, 8, L), _NEG, dtype=jnp.float32)
    iz = jnp.zeros((RB, 8, L), dtype=jnp.int32)

    def body(g, carry):
        t1, t2, t3, i1, i2, i3 = carry
        x = r_ref[m, :, pl.ds(g * 8, 8), :]
        ix = base + g * (8 * L)
        c1 = x > t1
        c2 = x > t2
        c3 = x > t3
        n1 = jnp.maximum(t1, x)
        n2 = jnp.maximum(t2, jnp.minimum(t1, x))
        n3 = jnp.maximum(t3, jnp.minimum(t2, x))
        j1 = jnp.where(c1, ix, i1)
        j2 = jnp.where(c1, i1, jnp.where(c2, ix, i2))
        j3 = jnp.where(c2, i2, jnp.where(c3, ix, i3))
        return n1, n2, n3, j1, j2, j3

    t1, t2, t3, i1, i2, i3 = jax.lax.fori_loop(
        0, G, body, (neg, neg, neg, iz, iz, iz)
    )
    cv = jnp.concatenate([t1, t2, t3], axis=1)
    ci = jnp.concatenate([i1, i2, i3], axis=1)
    return cv, ci


def _merge_topk(cv, ci, n):
    lvl_last = jax.lax.broadcasted_iota(jnp.int32, cv.shape, 1) >= (_D - 1) * 8
    ids = []
    unsafe = jnp.zeros((cv.shape[0], 1, 1), dtype=jnp.bool_)
    for _ in range(_K):
        mx = jnp.max(cv, axis=(1, 2), keepdims=True)
        am = jnp.min(jnp.where(cv == mx, ci, n), axis=(1, 2), keepdims=True)
        ids.append(am[:, 0])
        hit = jnp.logical_and(cv == mx, ci == am)
        unsafe = jnp.logical_or(
            unsafe,
            jnp.any(jnp.logical_and(hit, lvl_last), axis=(1, 2), keepdims=True),
        )
        cv = jnp.where(hit, _NEG, cv)
    return jnp.concatenate(ids, axis=1), unsafe


def _rec_kernel(w_ref, r_ref, out_ref):
    RB, S, L = out_ref.shape
    n = S * L

    cv0, ci0 = _online_top3(r_ref, 0, RB)
    cv1, ci1 = _online_top3(r_ref, 1, RB)
    h0, u0 = _merge_topk(cv0, ci0, n)
    h1, u1 = _merge_topk(cv1, ci1, n)
    unsafe = jnp.any(jnp.logical_or(u0, u1))

    giota = (
        jax.lax.broadcasted_iota(jnp.int32, (RB, S, L), 1) * L
        + jax.lax.broadcasted_iota(jnp.int32, (RB, S, L), 2)
    )
    x0 = r_ref[0]
    x1 = r_ref[1]

    ids0, ids1 = jax.lax.cond(
        unsafe,
        lambda: (_topk_ids_naive(x0, giota, n), _topk_ids_naive(x1, giota, n)),
        lambda: (h0, h1),
    )

    eq = ids0[:, :, None] == ids1[:, None, :]
    dup0 = jnp.sum(eq.astype(jnp.int32), axis=2)
    dup1 = jnp.sum(eq.astype(jnp.int32), axis=1)
    cand_ids = jnp.concatenate([ids0, ids1], axis=1)
    cnt = jnp.concatenate([1 + dup0, 1 + dup1], axis=1)
    valid = jnp.concatenate([jnp.ones_like(dup1), 1 - dup1], axis=1)
    key = jnp.where(valid > 0, (cnt << _IDBITS) - cand_ids, -_BIGI)
    jidx = jax.lax.broadcasted_iota(jnp.int32, key.shape, 1)

    sels = []
    for _ in range(_K):
        mk = jnp.max(key, axis=1, keepdims=True)
        amj = jnp.min(jnp.where(key == mk, jidx, 2 * _K), axis=1, keepdims=True)
        sels.append(jnp.sum(jnp.where(jidx == amj, cand_ids, 0), axis=1, keepdims=True))
        key = jnp.where(jidx == amj, -_BIGI, key)

    selmask = giota == sels[0][:, :, None]
    for s in sels[1:]:
        selmask = jnp.logical_or(selmask, giota == s[:, :, None])

    w0 = w_ref[0, 0]
    w1 = w_ref[0, 1]
    out_ref[...] = jnp.where(selmask, w0 * x0 + w1 * x1, _FILL)


def kernel(ratings, w):
    M, B, N = ratings.shape
    RB = 8
    S = N // _L
    r4 = ratings.reshape(M, B, S, _L)
    w2 = w.reshape(1, M).astype(jnp.float32)
    out = pl.pallas_call(
        _rec_kernel,
        grid=(B // RB,),
        in_specs=[
            pl.BlockSpec(memory_space=pltpu.SMEM),
            pl.BlockSpec((M, RB, S, _L), lambda i: (0, i, 0, 0)),
        ],
        out_specs=pl.BlockSpec((RB, S, _L), lambda i: (i, 0, 0)),
        out_shape=jax.ShapeDtypeStruct((B, S, _L), jnp.float32),
    )(w2, r4)
    return out.reshape(B, N)


# speedup vs baseline: 2.2698x; 1.2155x over previous
import jax
import jax.numpy as jnp
from jax.experimental import pallas as pl
from jax.experimental.pallas import tpu as pltpu

_K = 10
_L = 125
_FILL = float(jnp.finfo(jnp.float32).min)
_NEG = float("-inf")
_IDBITS = 17
_BIGP = 1 << 17
_BIGI = 1 << 30


def _topk_ids_naive(x, piota, big):
    work = x
    ids = []
    for _ in range(_K):
        mx = jnp.max(work, axis=(1, 2), keepdims=True)
        am = jnp.min(jnp.where(work == mx, piota, big), axis=(1, 2), keepdims=True)
        ids.append(am[:, 0])
        work = jnp.where(piota == am, _NEG, work)
    return jnp.concatenate(ids, axis=1)


def _online_top2s(r_ref, m, RB):
    S, L = r_ref.shape[2], r_ref.shape[3]
    G = S // 8
    base = (
        (jax.lax.broadcasted_iota(jnp.int32, (RB, 8, L), 1) << 7)
        + jax.lax.broadcasted_iota(jnp.int32, (RB, 8, L), 2)
    )
    neg = jnp.full((RB, 8, L), _NEG, dtype=jnp.float32)
    iz = jnp.zeros((RB, 8, L), dtype=jnp.int32)

    def body(g, carry):
        t1, t2, t3, i1, i2 = carry
        x = r_ref[m, :, pl.ds(g * 8, 8), :]
        ix = base + (g << 10)
        c1 = x > t1
        c2 = x > t2
        n1 = jnp.maximum(t1, x)
        n2 = jnp.maximum(t2, jnp.minimum(t1, x))
        n3 = jnp.maximum(t3, jnp.minimum(t2, x))
        j1 = jnp.where(c1, ix, i1)
        j2 = jnp.where(c1, i1, jnp.where(c2, ix, i2))
        return n1, n2, n3, j1, j2

    t1, t2, t3, i1, i2 = jax.lax.fori_loop(0, G, body, (neg, neg, neg, iz, iz))
    cv = jnp.concatenate([t1, t2, t3], axis=1)
    ci = jnp.concatenate([i1, i2, jnp.full_like(iz, _BIGP)], axis=1)
    return cv, ci


def _merge_topk(cv, ci):
    ids = []
    unsafe = jnp.zeros((cv.shape[0], 1, 1), dtype=jnp.bool_)
    for _ in range(_K):
        mx = jnp.max(cv, axis=(1, 2), keepdims=True)
        am = jnp.min(jnp.where(cv == mx, ci, _BIGI), axis=(1, 2), keepdims=True)
        ids.append(am[:, 0])
        unsafe = jnp.logical_or(unsafe, am >= _BIGP)
        cv = jnp.where(jnp.logical_and(cv == mx, ci == am), _NEG, cv)
    return jnp.concatenate(ids, axis=1), unsafe


def _rec_kernel(w_ref, r_ref, out_ref):
    RB, S, L = out_ref.shape

    cv0, ci0 = _online_top2s(r_ref, 0, RB)
    cv1, ci1 = _online_top2s(r_ref, 1, RB)
    h0, u0 = _merge_topk(cv0, ci0)
    h1, u1 = _merge_topk(cv1, ci1)
    unsafe = jnp.any(jnp.logical_or(u0, u1))

    x0 = r_ref[0]
    x1 = r_ref[1]

    def fallback():
        piota = (
            (jax.lax.broadcasted_iota(jnp.int32, (RB, S, L), 1) << 7)
            + jax.lax.broadcasted_iota(jnp.int32, (RB, S, L), 2)
        )
        return (
            _topk_ids_naive(x0, piota, _BIGP),
            _topk_ids_naive(x1, piota, _BIGP),
        )

    ids0, ids1 = jax.lax.cond(unsafe, fallback, lambda: (h0, h1))

    eq = ids0[:, :, None] == ids1[:, None, :]
    dup0 = jnp.sum(eq.astype(jnp.int32), axis=2)
    dup1 = jnp.sum(eq.astype(jnp.int32), axis=1)
    cand_ids = jnp.concatenate([ids0, ids1], axis=1)
    cnt = jnp.concatenate([1 + dup0, 1 + dup1], axis=1)
    valid = jnp.concatenate([jnp.ones_like(dup1), 1 - dup1], axis=1)
    key = jnp.where(valid > 0, (cnt << _IDBITS) - cand_ids, -_BIGI)
    jidx = jax.lax.broadcasted_iota(jnp.int32, key.shape, 1)

    sels = []
    for _ in range(_K):
        mk = jnp.max(key, axis=1, keepdims=True)
        amj = jnp.min(jnp.where(key == mk, jidx, 2 * _K), axis=1, keepdims=True)
        sels.append(jnp.sum(jnp.where(jidx == amj, cand_ids, 0), axis=1, keepdims=True))
        key = jnp.where(jidx == amj, -_BIGI, key)

    siota = jax.lax.broadcasted_iota(jnp.int32, (RB, S), 1)
    liota = jax.lax.broadcasted_iota(jnp.int32, (RB, L), 1)
    rowcode = jnp.zeros((RB, S), dtype=jnp.int32)
    lanecode = jnp.zeros((RB, L), dtype=jnp.int32)
    for k, s in enumerate(sels):
        rowcode = rowcode | jnp.where(siota == (s >> 7), 1 << k, 0)
        lanecode = lanecode | jnp.where(liota == (s & 127), 1 << k, 0)

    code = rowcode[:, :, None] & lanecode[:, None, :]
    w0 = w_ref[0, 0]
    w1 = w_ref[0, 1]
    out_ref[...] = jnp.where(code != 0, w0 * x0 + w1 * x1, _FILL)


def kernel(ratings, w):
    M, B, N = ratings.shape
    RB = 8
    S = N // _L
    r4 = ratings.reshape(M, B, S, _L)
    w2 = w.reshape(1, M).astype(jnp.float32)
    out = pl.pallas_call(
        _rec_kernel,
        grid=(B // RB,),
        in_specs=[
            pl.BlockSpec(memory_space=pltpu.SMEM),
            pl.BlockSpec((M, RB, S, _L), lambda i: (0, i, 0, 0)),
        ],
        out_specs=pl.BlockSpec((RB, S, _L), lambda i: (i, 0, 0)),
        out_shape=jax.ShapeDtypeStruct((B, S, _L), jnp.float32),
    )(w2, r4)
    return out.reshape(B, N)
